# baseline (speedup 1.0000x reference)
import functools

import jax
import jax.numpy as jnp
from jax import lax
from jax.experimental import pallas as pl
from jax.experimental.pallas import tpu as pltpu
from jax.experimental.pallas import tpu_sc as plsc

_B = 16384
_D = 64
_NC = 2
_NS = 16
_NW = _NC * _NS
_BPW = _B // _NW
_CHUNK = 128
_NCHUNK = _BPW // _CHUNK
_L = 16
_GPC = _CHUNK // _L

_mesh = plsc.VectorSubcoreMesh(core_axis_name="c", subcore_axis_name="s")


@functools.partial(
    pl.kernel,
    out_type=jax.ShapeDtypeStruct((_NW, _BPW), jnp.float32),
    mesh=_mesh,
    compiler_params=pltpu.CompilerParams(
        needs_layout_passes=False,
        use_tc_tiling_on_sc=False,
    ),
    scratch_types=[
        pltpu.VMEM((_NCHUNK, _CHUNK), jnp.int32),
        pltpu.VMEM((_NCHUNK, _CHUNK), jnp.int32),
        pltpu.VMEM((_NCHUNK, _CHUNK), jnp.int32),
        pltpu.VMEM((_NCHUNK, _CHUNK), jnp.int32),
        pltpu.VMEM((2, _CHUNK, _D + 8), jnp.float32),
        pltpu.VMEM((2, _CHUNK, _D + 8), jnp.float32),
        pltpu.VMEM((2, _CHUNK, 8), jnp.float32),
        pltpu.VMEM((2, _CHUNK, 8), jnp.float32),
        pltpu.VMEM((_BPW,), jnp.float32),
        pltpu.SemaphoreType.DMA,
    ],
)
def _sc_kernel(uid_hbm, iid_hbm, uemb_hbm, iemb_hbm, ubias_hbm, ibias_hbm,
               out_hbm, uidx, iidx, ubdx, ibdx,
               ubuf, ibuf, ubb, ibb, outv, sem):
    wid = lax.axis_index("s") * _NC + lax.axis_index("c")

    pltpu.sync_copy(uid_hbm.at[wid], uidx)
    pltpu.sync_copy(iid_hbm.at[wid], iidx)

    for c in range(_NCHUNK):
        for j in range(_GPC):
            sl = pl.ds(j * _L, _L)
            ubdx[c, sl] = lax.shift_right_logical(uidx[c, sl], 3)
            ibdx[c, sl] = lax.shift_right_logical(iidx[c, sl], 3)

    def fire(c):
        slot = c % 2
        return [
            pltpu.async_copy(uemb_hbm.at[uidx.at[c]], ubuf.at[slot], sem),
            pltpu.async_copy(iemb_hbm.at[iidx.at[c]], ibuf.at[slot], sem),
            pltpu.async_copy(ubias_hbm.at[ubdx.at[c]], ubb.at[slot], sem),
            pltpu.async_copy(ibias_hbm.at[ibdx.at[c]], ibb.at[slot], sem),
        ]

    iota = lax.iota(jnp.int32, _L)

    pending = fire(0)
    for c in range(_NCHUNK):
        for cp in pending:
            cp.wait()
        if c + 1 < _NCHUNK:
            pending = fire(c + 1)
        slot = c % 2
        ub_c = ubuf.at[slot]
        ib_c = ibuf.at[slot]
        ubb_c = ubb.at[slot]
        ibb_c = ibb.at[slot]

        def body(g, carry, c=c, ub_c=ub_c, ib_c=ib_c, ubb_c=ubb_c, ibb_c=ibb_c):
            rowk = g * _L + iota
            su = uidx[c, pl.ds(g * _L, _L)]
            si = iidx[c, pl.ds(g * _L, _L)]
            acc = (plsc.load_gather(ubb_c, [rowk, jnp.bitwise_and(su, 7)])
                   + plsc.load_gather(ibb_c, [rowk, jnp.bitwise_and(si, 7)]))
            col = jnp.full((_L,), 0, jnp.int32)
            for d in range(_D):
                cd = jnp.full((_L,), d, jnp.int32)
                pu = plsc.load_gather(ub_c, [rowk, cd])
                pi = plsc.load_gather(ib_c, [rowk, cd])
                acc = acc + pu * pi
            outv[pl.ds(c * _CHUNK + g * _L, _L)] = acc
            return carry

        lax.fori_loop(0, _GPC, body, 0)

    pltpu.sync_copy(outv, out_hbm.at[wid])


def kernel(user_ids, item_ids, user_emb, item_emb, user_bias, item_bias):
    uid = user_ids.astype(jnp.int32).reshape(_NW, _NCHUNK, _CHUNK)
    iid = item_ids.astype(jnp.int32).reshape(_NW, _NCHUNK, _CHUNK)
    ue = jnp.pad(user_emb, ((0, 0), (0, 8)))
    ie = jnp.pad(item_emb, ((0, 0), (0, 8)))
    ub = user_bias.reshape(-1, 8)
    ib = item_bias.reshape(-1, 8)
    out = _sc_kernel(uid, iid, ue, ie, ub, ib)
    return out.reshape(_B)


# speedup vs baseline: 1.9861x; 1.9861x over previous
import functools

import jax
import jax.numpy as jnp
from jax import lax
from jax.experimental import pallas as pl
from jax.experimental.pallas import tpu as pltpu
from jax.experimental.pallas import tpu_sc as plsc

_B = 16384
_D = 64
_NC = 2
_NS = 16
_NW = _NC * _NS
_BPW = _B // _NW
_CHUNK = 128
_NCHUNK = _BPW // _CHUNK
_L = 16
_GPC = _CHUNK // _L

_mesh = plsc.VectorSubcoreMesh(core_axis_name="c", subcore_axis_name="s")


@functools.partial(
    pl.kernel,
    out_type=jax.ShapeDtypeStruct((_NW, _BPW), jnp.float32),
    mesh=_mesh,
    compiler_params=pltpu.CompilerParams(
        needs_layout_passes=False,
        use_tc_tiling_on_sc=False,
    ),
    scratch_types=[
        pltpu.VMEM((_NCHUNK, _CHUNK), jnp.int32),
        pltpu.VMEM((_NCHUNK, _CHUNK), jnp.int32),
        pltpu.VMEM((_NCHUNK, _CHUNK), jnp.int32),
        pltpu.VMEM((_NCHUNK, _CHUNK), jnp.int32),
        pltpu.VMEM((2, _CHUNK, 2 * _D), jnp.float32),
        pltpu.VMEM((2, _CHUNK, 2 * _D), jnp.float32),
        pltpu.VMEM((2, _CHUNK, 8), jnp.float32),
        pltpu.VMEM((2, _CHUNK, 8), jnp.float32),
        pltpu.VMEM((_BPW,), jnp.float32),
        pltpu.SemaphoreType.DMA,
    ],
)
def _sc_kernel(uid_hbm, iid_hbm, uemb_hbm, iemb_hbm, ubias_hbm, ibias_hbm,
               out_hbm, uidx, iidx, ubdx, ibdx,
               ubuf, ibuf, ubb, ibb, outv, sem):
    wid = lax.axis_index("s") * _NC + lax.axis_index("c")

    pltpu.sync_copy(uid_hbm.at[wid], uidx)
    pltpu.sync_copy(iid_hbm.at[wid], iidx)

    for c in range(_NCHUNK):
        for j in range(_GPC):
            sl = pl.ds(j * _L, _L)
            ubdx[c, sl] = lax.shift_right_logical(uidx[c, sl], 3)
            ibdx[c, sl] = lax.shift_right_logical(iidx[c, sl], 3)

    def fire(c):
        slot = c % 2
        return [
            pltpu.async_copy(uemb_hbm.at[uidx.at[c]], ubuf.at[slot], sem),
            pltpu.async_copy(iemb_hbm.at[iidx.at[c]], ibuf.at[slot], sem),
            pltpu.async_copy(ubias_hbm.at[ubdx.at[c]], ubb.at[slot], sem),
            pltpu.async_copy(ibias_hbm.at[ibdx.at[c]], ibb.at[slot], sem),
        ]

    iota = lax.iota(jnp.int32, _L)

    pending = fire(0)
    for c in range(_NCHUNK):
        for cp in pending:
            cp.wait()
        if c + 1 < _NCHUNK:
            pending = fire(c + 1)
        slot = c % 2
        ub_c = ubuf.at[slot]
        ib_c = ibuf.at[slot]
        ubb_c = ubb.at[slot]
        ibb_c = ibb.at[slot]

        def body(g, carry, c=c, ub_c=ub_c, ib_c=ib_c, ubb_c=ubb_c, ibb_c=ibb_c):
            rowk = g * _L + iota
            su = uidx[c, pl.ds(g * _L, _L)]
            si = iidx[c, pl.ds(g * _L, _L)]
            acc = (plsc.load_gather(ubb_c, [rowk, jnp.bitwise_and(su, 7)])
                   + plsc.load_gather(ibb_c, [rowk, jnp.bitwise_and(si, 7)]))
            col = jnp.full((_L,), 0, jnp.int32)
            for d in range(_D):
                cd = jnp.full((_L,), d, jnp.int32)
                pu = plsc.load_gather(ub_c, [rowk, cd])
                pi = plsc.load_gather(ib_c, [rowk, cd])
                acc = acc + pu * pi
            outv[pl.ds(c * _CHUNK + g * _L, _L)] = acc
            return carry

        lax.fori_loop(0, _GPC, body, 0)

    pltpu.sync_copy(outv, out_hbm.at[wid])


def kernel(user_ids, item_ids, user_emb, item_emb, user_bias, item_bias):
    uid = user_ids.astype(jnp.int32).reshape(_NW, _NCHUNK, _CHUNK)
    iid = item_ids.astype(jnp.int32).reshape(_NW, _NCHUNK, _CHUNK)
    ue = jnp.pad(user_emb, ((0, 0), (0, _D)))
    ie = jnp.pad(item_emb, ((0, 0), (0, _D)))
    ub = user_bias.reshape(-1, 8)
    ib = item_bias.reshape(-1, 8)
    out = _sc_kernel(uid, iid, ue, ie, ub, ib)
    return out.reshape(_B)
